# baseline (speedup 1.0000x reference)
import jax
import jax.numpy as jnp
from jax.experimental import pallas as pl
from jax.experimental.pallas import tpu as pltpu

_BM = 400
_BP = 2000


def _outer(f1, f2, f3, adj, w1, w2, w3, womega, ucol, wd1, wd2, wd3,
           emb_o, r1_o, r2_o, r3_o, h_scr, emb_scr, cnt,
           *, n, d1, d2, d3, dz, np_, nblk):
    f32 = jnp.float32
    b16 = jnp.bfloat16
    cnt[0] = 0

    def inner(f1b, f2b, f3b, adjb, embb, r1b, r2b, r3b):
        s = cnt[0]

        @pl.when(s < np_)
        def _proj():
            h1 = jnp.dot(f1b[...].astype(b16), w1[...].astype(b16), preferred_element_type=f32)
            h2 = jnp.dot(f2b[...].astype(b16), w2[...].astype(b16), preferred_element_type=f32)
            h3 = jnp.dot(f3b[...].astype(b16), w3[...].astype(b16), preferred_element_type=f32)
            h_scr[pl.ds(s * _BP, _BP), :] = jnp.concatenate([h1, h2, h3], axis=1).astype(b16)

        @pl.when((s >= np_) & (s < np_ + nblk))
        def _fuse():
            i = s - np_
            z = jnp.dot(adjb[...].astype(b16), h_scr[...], preferred_element_type=f32)
            zs = [z[:, k * dz:(k + 1) * dz] for k in range(3)]
            w = womega[...].astype(b16)
            u = ucol[...].astype(b16)
            ss = []
            for zk in zs:
                v = jnp.tanh(jnp.dot(zk.astype(b16), w, preferred_element_type=f32))
                ss.append(jnp.dot(v.astype(b16), u, preferred_element_type=f32))
            m = jnp.maximum(jnp.maximum(ss[0], ss[1]), ss[2])
            es = [jnp.exp(x - m) for x in ss]
            den = es[0] + es[1] + es[2]
            emb = (es[0] * zs[0] + es[1] * zs[1] + es[2] * zs[2]) / den
            emb_scr[pl.ds(i * _BM, _BM), :] = emb

        @pl.when(s >= np_ + nblk)
        def _dec():
            i = s - np_ - nblk
            ae = jnp.dot(adjb[...].astype(b16), emb_scr[...].astype(b16),
                         preferred_element_type=f32)
            aeb = ae.astype(b16)
            r1b[...] = jnp.dot(aeb, wd1[...].astype(b16), preferred_element_type=f32)
            r2b[...] = jnp.dot(aeb, wd2[...].astype(b16), preferred_element_type=f32)
            r3b[...] = jnp.dot(aeb, wd3[...].astype(b16), preferred_element_type=f32)
            embb[...] = emb_scr[pl.ds(i * _BM, _BM), :]

        cnt[0] = s + 1

    def mov(s):
        return (jnp.where(s < np_, s, np_ - 1), 0)

    def madj(s):
        return (jnp.where(s < np_, 0,
                jnp.where(s < np_ + nblk, s - np_, s - np_ - nblk)), 0)

    def mout(s):
        return (jnp.where(s < np_ + nblk, 0, s - np_ - nblk), 0)

    pipe = pltpu.emit_pipeline(
        inner,
        grid=(np_ + 2 * nblk,),
        in_specs=[pl.BlockSpec((_BP, d1), mov,
                               pipeline_mode=pl.Buffered(buffer_count=1)),
                  pl.BlockSpec((_BP, d2), mov,
                               pipeline_mode=pl.Buffered(buffer_count=1)),
                  pl.BlockSpec((_BP, d3), mov,
                               pipeline_mode=pl.Buffered(buffer_count=1)),
                  pl.BlockSpec((_BM, n), madj,
                               pipeline_mode=pl.Buffered(buffer_count=3))],
        out_specs=[pl.BlockSpec((_BM, dz), mout),
                   pl.BlockSpec((_BM, d1), mout),
                   pl.BlockSpec((_BM, d2), mout),
                   pl.BlockSpec((_BM, d3), mout)],
    )
    pipe(f1, f2, f3, adj, emb_o, r1_o, r2_o, r3_o)


def kernel(features_omics1, features_omics2, features_omics3, adj,
           W_enc1, W_enc2, W_enc3, W_dec1, W_dec2, W_dec3,
           w_omega, u_omega):
    import functools
    n, d1 = features_omics1.shape
    d2 = features_omics2.shape[1]
    d3 = features_omics3.shape[1]
    dz = W_enc1.shape[1]
    nblk = n // _BM
    np_ = n // _BP

    body = functools.partial(_outer, n=n, d1=d1, d2=d2, d3=d3, dz=dz,
                             np_=np_, nblk=nblk)

    any_spec = pl.BlockSpec(memory_space=pl.ANY)
    vmem_spec = pl.BlockSpec(memory_space=pltpu.MemorySpace.VMEM)

    emb, r1, r2, r3 = pl.pallas_call(
        body,
        in_specs=[any_spec, any_spec, any_spec, any_spec,
                  vmem_spec, vmem_spec, vmem_spec,
                  vmem_spec, vmem_spec,
                  vmem_spec, vmem_spec, vmem_spec],
        out_specs=[any_spec, any_spec, any_spec, any_spec],
        out_shape=[jax.ShapeDtypeStruct((n, dz), jnp.float32),
                   jax.ShapeDtypeStruct((n, d1), jnp.float32),
                   jax.ShapeDtypeStruct((n, d2), jnp.float32),
                   jax.ShapeDtypeStruct((n, d3), jnp.float32)],
        scratch_shapes=[pltpu.VMEM((n, 3 * dz), jnp.bfloat16),
                        pltpu.VMEM((n, dz), jnp.float32),
                        pltpu.SMEM((1,), jnp.int32)],
        compiler_params=pltpu.CompilerParams(vmem_limit_bytes=64 * 1024 * 1024),
    )(features_omics1, features_omics2, features_omics3, adj,
      W_enc1, W_enc2, W_enc3, w_omega, u_omega, W_dec1, W_dec2, W_dec3)

    return emb, r1, r2, r3


# speedup vs baseline: 1.0246x; 1.0246x over previous
import jax
import jax.numpy as jnp
from jax.experimental import pallas as pl
from jax.experimental.pallas import tpu as pltpu

_BM = 400
_BP = 2000


def _body(f1, f2, f3, adjt, adjb, w1, w2, w3, womega, ucol, wd1, wd2, wd3,
          emb_out, r1, r2, r3, h_scr, emb_scr, embbf_scr, *, np_, nblk):
    s = pl.program_id(0)
    bm = 2 * adjt.shape[0]
    f32 = jnp.float32
    b16 = jnp.bfloat16

    @pl.when(s < np_)
    def _proj():
        h1 = jnp.dot(f1[...].astype(b16), w1[...].astype(b16), preferred_element_type=f32)
        h2 = jnp.dot(f2[...].astype(b16), w2[...].astype(b16), preferred_element_type=f32)
        h3 = jnp.dot(f3[...].astype(b16), w3[...].astype(b16), preferred_element_type=f32)
        h_scr[pl.ds(s * _BP, _BP), :] = jnp.concatenate([h1, h2, h3], axis=1).astype(b16)

    @pl.when((s >= np_) & (s < np_ + nblk))
    def _fuse():
        i = s - np_
        z = jnp.concatenate(
            [jnp.dot(adjt[...].astype(b16), h_scr[...], preferred_element_type=f32),
             jnp.dot(adjb[...].astype(b16), h_scr[...], preferred_element_type=f32)],
            axis=0)
        dz = womega.shape[0]
        zs = [z[:, k * dz:(k + 1) * dz] for k in range(3)]
        w = womega[...].astype(b16)
        u = ucol[...].astype(b16)
        ss = []
        for zk in zs:
            v = jnp.tanh(jnp.dot(zk.astype(b16), w, preferred_element_type=f32))
            ss.append(jnp.dot(v.astype(b16), u, preferred_element_type=f32))
        m = jnp.maximum(jnp.maximum(ss[0], ss[1]), ss[2])
        es = [jnp.exp(x - m) for x in ss]
        den = es[0] + es[1] + es[2]
        emb = (es[0] * zs[0] + es[1] * zs[1] + es[2] * zs[2]) / den
        emb_scr[pl.ds(i * bm, bm), :] = emb
        embbf_scr[pl.ds(i * bm, bm), :] = emb.astype(b16)

    @pl.when(s >= np_ + nblk)
    def _dec():
        i = s - np_ - nblk
        ae = jnp.concatenate(
            [jnp.dot(adjt[...].astype(b16), embbf_scr[...], preferred_element_type=f32),
             jnp.dot(adjb[...].astype(b16), embbf_scr[...], preferred_element_type=f32)],
            axis=0)
        aeb = ae.astype(b16)
        r1[...] = jnp.dot(aeb, wd1[...].astype(b16), preferred_element_type=f32)
        r2[...] = jnp.dot(aeb, wd2[...].astype(b16), preferred_element_type=f32)
        r3[...] = jnp.dot(aeb, wd3[...].astype(b16), preferred_element_type=f32)
        emb_out[...] = emb_scr[pl.ds(i * bm, bm), :]


def kernel(features_omics1, features_omics2, features_omics3, adj,
           W_enc1, W_enc2, W_enc3, W_dec1, W_dec2, W_dec3,
           w_omega, u_omega):
    import functools
    n, d1 = features_omics1.shape
    d2 = features_omics2.shape[1]
    d3 = features_omics3.shape[1]
    dz = W_enc1.shape[1]
    nblk = n // _BM
    np_ = n // _BP

    def mov(s):
        return (jnp.where(s < np_, s, np_ - 1), 0)

    def madj(s):
        return (jnp.where(s < np_, 0,
                jnp.where(s < np_ + nblk, s - np_, s - np_ - nblk)), 0)

    def mout(s):
        return (jnp.where(s < np_ + nblk, 0, s - np_ - nblk), 0)

    def mconst(s):
        return (0, 0)

    body = functools.partial(_body, np_=np_, nblk=nblk)

    emb, r1, r2, r3 = pl.pallas_call(
        body,
        grid=(np_ + 2 * nblk,),
        in_specs=[pl.BlockSpec((_BP, d1), mov),
                  pl.BlockSpec((_BP, d2), mov),
                  pl.BlockSpec((_BP, d3), mov),
                  pl.BlockSpec((_BM // 2, n), lambda s: (2 * madj(s)[0], 0)),
                  pl.BlockSpec((_BM // 2, n), lambda s: (2 * madj(s)[0] + 1, 0)),
                  pl.BlockSpec((d1, dz), mconst),
                  pl.BlockSpec((d2, dz), mconst),
                  pl.BlockSpec((d3, dz), mconst),
                  pl.BlockSpec((dz, dz), mconst),
                  pl.BlockSpec((dz, 1), mconst),
                  pl.BlockSpec((dz, d1), mconst),
                  pl.BlockSpec((dz, d2), mconst),
                  pl.BlockSpec((dz, d3), mconst)],
        out_specs=[pl.BlockSpec((_BM, dz), mout),
                   pl.BlockSpec((_BM, d1), mout),
                   pl.BlockSpec((_BM, d2), mout),
                   pl.BlockSpec((_BM, d3), mout)],
        out_shape=[jax.ShapeDtypeStruct((n, dz), jnp.float32),
                   jax.ShapeDtypeStruct((n, d1), jnp.float32),
                   jax.ShapeDtypeStruct((n, d2), jnp.float32),
                   jax.ShapeDtypeStruct((n, d3), jnp.float32)],
        scratch_shapes=[pltpu.VMEM((n, 3 * dz), jnp.bfloat16),
                        pltpu.VMEM((n, dz), jnp.float32),
                        pltpu.VMEM((n, dz), jnp.bfloat16)],
    )(features_omics1, features_omics2, features_omics3, adj, adj,
      W_enc1, W_enc2, W_enc3, w_omega, u_omega, W_dec1, W_dec2, W_dec3)

    return emb, r1, r2, r3
